# Initial kernel scaffold; baseline (speedup 1.0000x reference)
import jax
import jax.numpy as jnp
from jax.experimental import pallas as pl


def kernel(x, gate_w, correction_bias, Wi, Wo, shared_Wi, shared_Wo):
    raise NotImplementedError("write your pallas kernel here")


# speedup vs baseline: 16.1818x; 16.1818x over previous
import functools

import jax
import jax.numpy as jnp
from jax.experimental import pallas as pl
from jax.experimental.pallas import tpu as pltpu

E = 64
TOP_K = 2
H = 768
I = 768
I2 = 2 * I
T = 8192
S = T * TOP_K
BK = 256
NP = S + E * BK
NB = NP // BK
BT = 512


def _router_shared_body(x_ref, gw_ref, cb_ref, swi_ref, swo_ref,
                        sh_ref, idx_ref, w_ref):
    xb = x_ref[...]
    h = jnp.dot(xb, swi_ref[...], preferred_element_type=jnp.float32)
    xp = h[:, :I]
    g = h[:, I:]
    act = g * jax.nn.sigmoid(g) * xp
    sh_ref[...] = jnp.dot(act, swo_ref[...], preferred_element_type=jnp.float32)
    logits = jax.nn.sigmoid(
        jax.lax.dot_general(xb, gw_ref[...],
                            (((1,), (1,)), ((), ())),
                            preferred_element_type=jnp.float32))
    sel = logits + cb_ref[...]
    iota = jax.lax.broadcasted_iota(jnp.int32, (BT, E), 1)
    m1 = jnp.max(sel, axis=1, keepdims=True)
    i1 = jnp.min(jnp.where(sel == m1, iota, E), axis=1, keepdims=True)
    w1 = jnp.max(jnp.where(iota == i1, logits, -1.0), axis=1, keepdims=True)
    sel2 = jnp.where(iota == i1, -jnp.inf, sel)
    m2 = jnp.max(sel2, axis=1, keepdims=True)
    i2 = jnp.min(jnp.where(sel2 == m2, iota, E), axis=1, keepdims=True)
    w2 = jnp.max(jnp.where(iota == i2, logits, -1.0), axis=1, keepdims=True)
    ws = w1 + w2
    idx_ref[...] = jnp.concatenate([i1, i2], axis=1)
    w_ref[...] = jnp.concatenate([w1 / ws, w2 / ws], axis=1)


def _router_shared(xf, gate_w, cb2, shared_Wi, shared_Wo):
    nblk = T // BT
    return pl.pallas_call(
        _router_shared_body,
        grid=(nblk,),
        in_specs=[
            pl.BlockSpec((BT, H), lambda b: (b, 0)),
            pl.BlockSpec((E, H), lambda b: (0, 0)),
            pl.BlockSpec((1, E), lambda b: (0, 0)),
            pl.BlockSpec((H, I2), lambda b: (0, 0)),
            pl.BlockSpec((I, H), lambda b: (0, 0)),
        ],
        out_specs=[
            pl.BlockSpec((BT, H), lambda b: (b, 0)),
            pl.BlockSpec((BT, TOP_K), lambda b: (b, 0)),
            pl.BlockSpec((BT, TOP_K), lambda b: (b, 0)),
        ],
        out_shape=[
            jax.ShapeDtypeStruct((T, H), jnp.float32),
            jax.ShapeDtypeStruct((T, TOP_K), jnp.int32),
            jax.ShapeDtypeStruct((T, TOP_K), jnp.float32),
        ],
    )(xf, gate_w, cb2, shared_Wi, shared_Wo)


def _grouped_mm_body(be_ref, xs_ref, wi_ref, wo_ref, ys_ref):
    xb = xs_ref[...]
    h = jnp.dot(xb, wi_ref[0], preferred_element_type=jnp.float32)
    xp = h[:, :I]
    g = h[:, I:]
    act = g * jax.nn.sigmoid(g) * xp
    ys_ref[...] = jnp.dot(act, wo_ref[0], preferred_element_type=jnp.float32)


def _grouped_mm(be, xs, Wi, Wo):
    return pl.pallas_call(
        _grouped_mm_body,
        grid_spec=pltpu.PrefetchScalarGridSpec(
            num_scalar_prefetch=1,
            grid=(NB,),
            in_specs=[
                pl.BlockSpec((BK, H), lambda b, be: (b, 0)),
                pl.BlockSpec((1, H, I2), lambda b, be: (be[b], 0, 0)),
                pl.BlockSpec((1, I, H), lambda b, be: (be[b], 0, 0)),
            ],
            out_specs=pl.BlockSpec((BK, H), lambda b, be: (b, 0)),
        ),
        out_shape=jax.ShapeDtypeStruct((NP, H), jnp.float32),
    )(be, xs, Wi, Wo)


def kernel(x, gate_w, correction_bias, Wi, Wo, shared_Wi, shared_Wo):
    orig_shape = x.shape
    xf = x.reshape(-1, H)
    cb2 = correction_bias.reshape(1, E)

    shared_out, idx, w = _router_shared(xf, gate_w, cb2, shared_Wi, shared_Wo)

    eids = idx.reshape(-1)
    counts = jnp.zeros((E,), jnp.int32).at[eids].add(1)
    nbl = (counts + BK - 1) // BK
    poff = jnp.concatenate([jnp.zeros((1,), jnp.int32),
                            jnp.cumsum(nbl * BK)[:-1].astype(jnp.int32)])
    uoff = jnp.concatenate([jnp.zeros((1,), jnp.int32),
                            jnp.cumsum(counts)[:-1].astype(jnp.int32)])
    order = jnp.argsort(eids, stable=True)
    se = eids[order]
    poso = poff[se] + (jnp.arange(S, dtype=jnp.int32) - uoff[se])
    pos = jnp.zeros((S,), jnp.int32).at[order].set(poso)
    src_tok = jnp.zeros((NP,), jnp.int32).at[poso].set(order // TOP_K)
    cumblk = jnp.cumsum(nbl).astype(jnp.int32)
    be = jnp.minimum(
        jnp.searchsorted(cumblk, jnp.arange(NB, dtype=jnp.int32), side="right"),
        E - 1).astype(jnp.int32)
    xs = xf[src_tok]

    ys = _grouped_mm(be, xs, Wi, Wo)

    r = ys[pos.reshape(T, TOP_K)]
    out = (r * w[..., None]).sum(axis=1) + shared_out
    return out.reshape(orig_shape)
